# Initial kernel scaffold; baseline (speedup 1.0000x reference)
import jax
import jax.numpy as jnp
from jax.experimental import pallas as pl


def kernel(utterance_features, audio_features, visual_features, semantic_adj, structure_adj, params):
    raise NotImplementedError("write your pallas kernel here")


# speedup vs baseline: 21.1885x; 21.1885x over previous
import functools
import jax
import jax.numpy as jnp
from jax.experimental import pallas as pl
from jax.experimental.pallas import tpu as pltpu

_EMB, _AEMB, _VEMB = 1024, 512, 512
_D, _E, _K = 256, 8, 2
_B, _N = 8, 256
_L, _NCLS = 2, 7
_F32 = jnp.float32


def _proj_body(u_ref, a_ref, v_ref, w1_ref, w2_ref, w3_ref,
               b1_ref, b2_ref, b3_ref, ht_ref, ha_ref, hv_ref):
    ht_ref[...] = jnp.maximum(
        jnp.dot(u_ref[...], w1_ref[...], preferred_element_type=_F32) + b1_ref[...], 0.0)
    ha_ref[...] = jnp.maximum(
        jnp.dot(a_ref[...], w2_ref[...], preferred_element_type=_F32) + b2_ref[...], 0.0)
    hv_ref[...] = jnp.maximum(
        jnp.dot(v_ref[...], w3_ref[...], preferred_element_type=_F32) + b3_ref[...], 0.0)


def _proj(u, a, v, p):
    u2 = u.reshape(_B * _N, _EMB)
    a2 = a.reshape(_B * _N, _AEMB)
    v2 = v.reshape(_B * _N, _VEMB)
    out = pl.pallas_call(
        _proj_body,
        grid=(_B,),
        in_specs=[
            pl.BlockSpec((_N, _EMB), lambda b: (b, 0)),
            pl.BlockSpec((_N, _AEMB), lambda b: (b, 0)),
            pl.BlockSpec((_N, _VEMB), lambda b: (b, 0)),
            pl.BlockSpec((_EMB, _D), lambda b: (0, 0)),
            pl.BlockSpec((_AEMB, _D), lambda b: (0, 0)),
            pl.BlockSpec((_VEMB, _D), lambda b: (0, 0)),
            pl.BlockSpec((1, _D), lambda b: (0, 0)),
            pl.BlockSpec((1, _D), lambda b: (0, 0)),
            pl.BlockSpec((1, _D), lambda b: (0, 0)),
        ],
        out_specs=[pl.BlockSpec((_N, _D), lambda b: (b, 0))] * 3,
        out_shape=[jax.ShapeDtypeStruct((_B * _N, _D), _F32)] * 3,
    )(u2, a2, v2, p['fc1_w'], p['fc2_w'], p['fc3_w'],
      p['fc1_b'].reshape(1, _D), p['fc2_b'].reshape(1, _D), p['fc3_b'].reshape(1, _D))
    return [o.reshape(_B, _N, _D) for o in out]


def _route_body(h_ref, gate_ref, w_ref, sums_ref, lb_ref):
    b = pl.program_id(0)
    logits = jnp.dot(h_ref[0], gate_ref[...], preferred_element_type=_F32)
    m = jnp.max(logits, axis=-1, keepdims=True)
    ex = jnp.exp(logits - m)
    probs = ex / jnp.sum(ex, axis=-1, keepdims=True)

    lane = jax.lax.broadcasted_iota(jnp.int32, (_N, _E), 1)
    m1 = jnp.max(probs, axis=-1, keepdims=True)
    i1 = jnp.argmax(probs, axis=-1)[:, None]
    is1 = lane == i1
    probs2 = jnp.where(is1, -1.0, probs)
    m2 = jnp.max(probs2, axis=-1, keepdims=True)
    i2 = jnp.argmax(probs2, axis=-1)[:, None]
    is2 = lane == i2
    t = m1 + m2 + 1e-9
    w_ref[0] = jnp.where(is1, m1 / t, 0.0) + jnp.where(is2, m2 / t, 0.0)

    ind = (is1 | is2).astype(_F32)

    @pl.when(b == 0)
    def _():
        sums_ref[...] = jnp.zeros_like(sums_ref)

    sums_ref[0:1, :] += jnp.sum(ind, axis=0, keepdims=True)
    sums_ref[1:2, :] += jnp.sum(probs, axis=0, keepdims=True)

    @pl.when(b == _B - 1)
    def _():
        frac = sums_ref[0:1, :] / (_B * _N * _K)
        mean_probs = sums_ref[1:2, :] / (_B * _N)
        lb_ref[...] = _E * jnp.sum(frac * mean_probs, axis=1, keepdims=True)


def _route(gp, h):
    w, _, lb = pl.pallas_call(
        _route_body,
        grid=(_B,),
        in_specs=[
            pl.BlockSpec((1, _N, _D), lambda b: (b, 0, 0)),
            pl.BlockSpec((_D, _E), lambda b: (0, 0)),
        ],
        out_specs=[
            pl.BlockSpec((1, _N, _E), lambda b: (b, 0, 0)),
            pl.BlockSpec((8, _E), lambda b: (0, 0)),
            pl.BlockSpec((1, 1), lambda b: (0, 0)),
        ],
        out_shape=[
            jax.ShapeDtypeStruct((_B, _N, _E), _F32),
            jax.ShapeDtypeStruct((8, _E), _F32),
            jax.ShapeDtypeStruct((1, 1), _F32),
        ],
    )(h, gp['gate'])
    return w, lb[0, 0]


def _attention(h, adj, W_ref, asrc_ref, adst_ref, rb_ref, num_rel):
    Wh = jnp.dot(h, W_ref[0], preferred_element_type=_F32)
    src = jnp.dot(Wh, asrc_ref[0].reshape(_D, 1),
                  preferred_element_type=_F32)
    dst = jax.lax.dot_general(adst_ref[0], Wh, (((1,), (1,)), ((), ())),
                              preferred_element_type=_F32)
    bias = jnp.zeros((_N, _N), _F32)
    for r in range(1, num_rel + 1):
        bias = bias + jnp.where(adj == r, rb_ref[0, 0, r], 0.0)
    scores = src + dst + bias
    scores = jnp.where(scores >= 0, scores, 0.2 * scores)
    mask = adj > 0
    scores = jnp.where(mask, scores, -1e9)
    rowmax = jnp.max(scores, axis=1, keepdims=True)
    ex = jnp.exp(scores - rowmax)
    att = ex / jnp.sum(ex, axis=1, keepdims=True)
    anyrow = jnp.max(mask.astype(_F32), axis=1, keepdims=True)
    return att * anyrow, Wh


def _wcol(w_ref, e):
    lane = jax.lax.broadcasted_iota(jnp.int32, (_N, _E), 1)
    return jnp.sum(jnp.where(lane == e, w_ref[0], 0.0), axis=1, keepdims=True)


def _elu(x):
    return jnp.where(x > 0, x, jnp.exp(x) - 1.0)


def _moe_single_body(h_ref, adj_ref, w_ref, W_ref, asrc_ref, adst_ref, rb_ref,
                     out_ref, *, num_rel):
    e = pl.program_id(1)
    att, Wh = _attention(h_ref[0], adj_ref[0], W_ref, asrc_ref, adst_ref,
                         rb_ref, num_rel)
    contrib = _wcol(w_ref, e) * jnp.dot(att, Wh, preferred_element_type=_F32)

    @pl.when(e == 0)
    def _():
        out_ref[0] = contrib

    @pl.when(e != 0)
    def _():
        out_ref[0] += contrib

    @pl.when(e == _E - 1)
    def _():
        out_ref[0] = _elu(out_ref[0])


def _moe_pair_body(h_ref, b1_ref, b2_ref, adj_ref, w_ref, W_ref, Wv_ref,
                   asrc_ref, adst_ref, rb_ref, o1_ref, o2_ref, *, num_rel):
    e = pl.program_id(1)
    att, _ = _attention(h_ref[0], adj_ref[0], W_ref, asrc_ref, adst_ref,
                        rb_ref, num_rel)
    wc = _wcol(w_ref, e)
    V1 = jnp.dot(b1_ref[0], Wv_ref[0], preferred_element_type=_F32)
    V2 = jnp.dot(b2_ref[0], Wv_ref[0], preferred_element_type=_F32)
    c1 = wc * jnp.dot(att, V1, preferred_element_type=_F32)
    c2 = wc * jnp.dot(att, V2, preferred_element_type=_F32)

    @pl.when(e == 0)
    def _():
        o1_ref[0] = c1
        o2_ref[0] = c2

    @pl.when(e != 0)
    def _():
        o1_ref[0] += c1
        o2_ref[0] += c2

    @pl.when(e == _E - 1)
    def _():
        o1_ref[0] = _elu(o1_ref[0])
        o2_ref[0] = _elu(o2_ref[0])


def _gat_specs(num_rel):
    return [
        pl.BlockSpec((1, _D, _D), lambda b, e: (e, 0, 0)),
        pl.BlockSpec((1, 1, _D), lambda b, e: (e, 0, 0)),
        pl.BlockSpec((1, 1, _D), lambda b, e: (e, 0, 0)),
        pl.BlockSpec((1, 1, num_rel + 1), lambda b, e: (e, 0, 0)),
    ]


def _gat_vecs(gp, num_rel):
    return (gp['a_src'].reshape(_E, 1, _D), gp['a_dst'].reshape(_E, 1, _D),
            gp['rel_bias'].reshape(_E, 1, num_rel + 1))


def _moe_single(gp, h, adj, w, num_rel):
    out = pl.pallas_call(
        functools.partial(_moe_single_body, num_rel=num_rel),
        grid=(_B, _E),
        in_specs=[
            pl.BlockSpec((1, _N, _D), lambda b, e: (b, 0, 0)),
            pl.BlockSpec((1, _N, _N), lambda b, e: (b, 0, 0)),
            pl.BlockSpec((1, _N, _E), lambda b, e: (b, 0, 0)),
        ] + _gat_specs(num_rel),
        out_specs=pl.BlockSpec((1, _N, _D), lambda b, e: (b, 0, 0)),
        out_shape=jax.ShapeDtypeStruct((_B, _N, _D), _F32),
        compiler_params=pltpu.CompilerParams(
            dimension_semantics=("arbitrary", "arbitrary")),
    )(h, adj, w, gp['W'], *_gat_vecs(gp, num_rel))
    return out


def _moe_pair(gp, h, b1, b2, adj, w, num_rel):
    o1, o2 = pl.pallas_call(
        functools.partial(_moe_pair_body, num_rel=num_rel),
        grid=(_B, _E),
        in_specs=[
            pl.BlockSpec((1, _N, _D), lambda b, e: (b, 0, 0)),
            pl.BlockSpec((1, _N, _D), lambda b, e: (b, 0, 0)),
            pl.BlockSpec((1, _N, _D), lambda b, e: (b, 0, 0)),
            pl.BlockSpec((1, _N, _N), lambda b, e: (b, 0, 0)),
            pl.BlockSpec((1, _N, _E), lambda b, e: (b, 0, 0)),
            pl.BlockSpec((1, _D, _D), lambda b, e: (e, 0, 0)),
            pl.BlockSpec((1, _D, _D), lambda b, e: (e, 0, 0)),
            pl.BlockSpec((1, 1, _D), lambda b, e: (e, 0, 0)),
            pl.BlockSpec((1, 1, _D), lambda b, e: (e, 0, 0)),
            pl.BlockSpec((1, 1, num_rel + 1), lambda b, e: (e, 0, 0)),
        ],
        out_specs=[pl.BlockSpec((1, _N, _D), lambda b, e: (b, 0, 0))] * 2,
        out_shape=[jax.ShapeDtypeStruct((_B, _N, _D), _F32)] * 2,
        compiler_params=pltpu.CompilerParams(
            dimension_semantics=("arbitrary", "arbitrary")),
    )(h, b1, b2, adj, w, gp['W'], gp['Wv'], *_gat_vecs(gp, num_rel))
    return o1, o2


def _affine_pair_body(h1_ref, h2_ref, w1_ref, w2_ref, o1_ref, o2_ref):
    h1 = h1_ref[0]
    h2 = h2_ref[0]

    def one(a, wf, bmat):
        s = jax.lax.dot_general(
            jnp.dot(a, wf, preferred_element_type=_F32), bmat,
            (((1,), (1,)), ((), ())), preferred_element_type=_F32)
        s = s - jnp.max(s, axis=1, keepdims=True)
        ex = jnp.exp(s)
        att = ex / jnp.sum(ex, axis=1, keepdims=True)
        return jnp.dot(att, bmat, preferred_element_type=_F32)

    o1_ref[0] = one(h1, w1_ref[...], h2)
    o2_ref[0] = one(h2, w2_ref[...], h1)


def _affine_pair(h1, h2, w1, w2):
    return pl.pallas_call(
        _affine_pair_body,
        grid=(_B,),
        in_specs=[
            pl.BlockSpec((1, _N, _D), lambda b: (b, 0, 0)),
            pl.BlockSpec((1, _N, _D), lambda b: (b, 0, 0)),
            pl.BlockSpec((_D, _D), lambda b: (0, 0)),
            pl.BlockSpec((_D, _D), lambda b: (0, 0)),
        ],
        out_specs=[pl.BlockSpec((1, _N, _D), lambda b: (b, 0, 0))] * 2,
        out_shape=[jax.ShapeDtypeStruct((_B, _N, _D), _F32)] * 2,
    )(h1, h2, w1, w2)


def _mlp_body(f1_ref, f2_ref, f3_ref, f4_ref, f5_ref, f6_ref,
              u_ref, a_ref, v_ref, w1_ref, w2_ref, w3_ref,
              b1_ref, b2_ref, b3_ref, out_ref):
    parts = [f1_ref[0], f2_ref[0], f3_ref[0], f4_ref[0], f5_ref[0], f6_ref[0],
             u_ref[0], a_ref[0], v_ref[0]]
    offs = [0, _D, 2 * _D, 3 * _D, 4 * _D, 5 * _D,
            6 * _D, 6 * _D + _EMB, 6 * _D + _EMB + _AEMB]
    x = b1_ref[...]
    for part, off in zip(parts, offs):
        x = x + jnp.dot(part, w1_ref[off:off + part.shape[1], :],
                        preferred_element_type=_F32)
    x = jnp.maximum(x, 0.0)
    x = jnp.maximum(jnp.dot(x, w2_ref[...], preferred_element_type=_F32)
                    + b2_ref[...], 0.0)
    out_ref[0] = jnp.dot(x, w3_ref[...], preferred_element_type=_F32) + b3_ref[...]


def _mlp(feats, u, a, v, p):
    fin = 6 * _D + _EMB + _AEMB + _VEMB
    out = pl.pallas_call(
        _mlp_body,
        grid=(_B,),
        in_specs=[pl.BlockSpec((1, _N, _D), lambda b: (b, 0, 0))] * 6 + [
            pl.BlockSpec((1, _N, _EMB), lambda b: (b, 0, 0)),
            pl.BlockSpec((1, _N, _AEMB), lambda b: (b, 0, 0)),
            pl.BlockSpec((1, _N, _VEMB), lambda b: (b, 0, 0)),
            pl.BlockSpec((fin, _D), lambda b: (0, 0)),
            pl.BlockSpec((_D, _D), lambda b: (0, 0)),
            pl.BlockSpec((_D, _NCLS), lambda b: (0, 0)),
            pl.BlockSpec((1, _D), lambda b: (0, 0)),
            pl.BlockSpec((1, _D), lambda b: (0, 0)),
            pl.BlockSpec((1, _NCLS), lambda b: (0, 0)),
        ],
        out_specs=pl.BlockSpec((1, _N, _NCLS), lambda b: (b, 0, 0)),
        out_shape=jax.ShapeDtypeStruct((_B, _N, _NCLS), _F32),
    )(*feats, u, a, v, p['mlp_w'][0], p['mlp_w'][1], p['mlp_w'][2],
      p['mlp_b'][0].reshape(1, _D), p['mlp_b'][1].reshape(1, _D),
      p['mlp_b'][2].reshape(1, _NCLS))
    return out


def kernel(utterance_features, audio_features, visual_features,
           semantic_adj, structure_adj, params):
    p = params
    h_t, h_a, h_v = _proj(utterance_features, audio_features,
                          visual_features, p)
    H = [[h_t, h_a, h_v]]
    lb_total = jnp.float32(0.0)
    for l in range(_L):
        sem_src = H[0] if l == 0 else H[2 * l - 1]
        stu_src = H[0] if l == 0 else H[2 * l]

        w_spk, lb1 = _route(p['spk'][l], sem_src[0])
        h_sem = _moe_single(p['spk'][l], sem_src[0], semantic_adj, w_spk, 6)
        w_dis, lb2 = _route(p['dis'][l], stu_src[0])
        h_stu = _moe_single(p['dis'][l], stu_src[0], structure_adj, w_dis, 18)
        w_ms, lb3 = _route(p['mspk'][l], sem_src[0])
        h_sta, h_stv = _moe_pair(p['mspk'][l], sem_src[0], sem_src[1],
                                 sem_src[2], semantic_adj, w_ms, 6)
        w_md, lb5 = _route(p['mdis'][l], stu_src[0])
        h_dta, h_dtv = _moe_pair(p['mdis'][l], stu_src[0], stu_src[1],
                                 stu_src[2], structure_adj, w_md, 18)
        lb_total = lb_total + lb1 + lb2 + 2.0 * lb3 + 2.0 * lb5

        o1, o2 = _affine_pair(h_sem, h_stu, p['affine1'], p['affine2'])
        o7, o8 = _affine_pair(h_sta, h_dta, p['affine7'], p['affine8'])
        o9, o10 = _affine_pair(h_stv, h_dtv, p['affine9'], p['affine10'])
        H.append([o1, o7, o9])
        H.append([o2, o8, o10])

    feats = [H[-2][0], H[-1][0], H[-2][1], H[-1][1], H[-2][2], H[-1][2]]
    x = _mlp(feats, utterance_features, audio_features,
             visual_features, p)
    return x, lb_total


# speedup vs baseline: 23.8291x; 1.1246x over previous
import functools
import jax
import jax.numpy as jnp
from jax.experimental import pallas as pl
from jax.experimental.pallas import tpu as pltpu

_EMB, _AEMB, _VEMB = 1024, 512, 512
_D, _E, _K = 256, 8, 2
_B, _N = 8, 256
_L, _NCLS = 2, 7
_F32 = jnp.float32
_BF16 = jnp.bfloat16


def _bf(x):
    return x.astype(_BF16)


def _elu(x):
    return jnp.where(x > 0, x, jnp.exp(x) - 1.0)


def _softmax_rows(s):
    m = jnp.max(s, axis=1, keepdims=True)
    ex = jnp.exp(s - m)
    return ex / jnp.sum(ex, axis=1, keepdims=True)


def _proj_body(u_ref, a_ref, v_ref, w1_ref, w2_ref, w3_ref,
               b1_ref, b2_ref, b3_ref, ht_ref, ha_ref, hv_ref):
    ht_ref[...] = jnp.maximum(
        jnp.dot(u_ref[...], w1_ref[...], preferred_element_type=_F32) + b1_ref[...], 0.0)
    ha_ref[...] = jnp.maximum(
        jnp.dot(a_ref[...], w2_ref[...], preferred_element_type=_F32) + b2_ref[...], 0.0)
    hv_ref[...] = jnp.maximum(
        jnp.dot(v_ref[...], w3_ref[...], preferred_element_type=_F32) + b3_ref[...], 0.0)


def _proj(u, a, v, p):
    u2 = _bf(u.reshape(_B * _N, _EMB))
    a2 = _bf(a.reshape(_B * _N, _AEMB))
    v2 = _bf(v.reshape(_B * _N, _VEMB))
    out = pl.pallas_call(
        _proj_body,
        grid=(_B,),
        in_specs=[
            pl.BlockSpec((_N, _EMB), lambda b: (b, 0)),
            pl.BlockSpec((_N, _AEMB), lambda b: (b, 0)),
            pl.BlockSpec((_N, _VEMB), lambda b: (b, 0)),
            pl.BlockSpec((_EMB, _D), lambda b: (0, 0)),
            pl.BlockSpec((_AEMB, _D), lambda b: (0, 0)),
            pl.BlockSpec((_VEMB, _D), lambda b: (0, 0)),
            pl.BlockSpec((1, _D), lambda b: (0, 0)),
            pl.BlockSpec((1, _D), lambda b: (0, 0)),
            pl.BlockSpec((1, _D), lambda b: (0, 0)),
        ],
        out_specs=[pl.BlockSpec((_N, _D), lambda b: (b, 0))] * 3,
        out_shape=[jax.ShapeDtypeStruct((_B * _N, _D), _F32)] * 3,
    )(u2, a2, v2, _bf(p['fc1_w']), _bf(p['fc2_w']), _bf(p['fc3_w']),
      p['fc1_b'].reshape(1, _D), p['fc2_b'].reshape(1, _D), p['fc3_b'].reshape(1, _D))
    return [o.reshape(_B, _N, _D) for o in out]


def _topk_w(probs):
    lane = jax.lax.broadcasted_iota(jnp.int32, (_N, _E), 1)
    m1 = jnp.max(probs, axis=-1, keepdims=True)
    i1 = jnp.argmax(probs, axis=-1)[:, None]
    is1 = lane == i1
    probs2 = jnp.where(is1, -1.0, probs)
    m2 = jnp.max(probs2, axis=-1, keepdims=True)
    i2 = jnp.argmax(probs2, axis=-1)[:, None]
    is2 = lane == i2
    t = m1 + m2 + 1e-9
    w = jnp.where(is1, m1 / t, 0.0) + jnp.where(is2, m2 / t, 0.0)
    return w, (is1 | is2).astype(_F32)


def _route2_body(h_ref, gates_ref, w1_ref, w2_ref, sums_ref, lb_ref):
    b = pl.program_id(0)
    logits = jnp.dot(h_ref[0], gates_ref[...], preferred_element_type=_F32)

    @pl.when(b == 0)
    def _():
        sums_ref[...] = jnp.zeros_like(sums_ref)

    for k, w_ref in ((0, w1_ref), (1, w2_ref)):
        lg = logits[:, k * _E:(k + 1) * _E]
        m = jnp.max(lg, axis=-1, keepdims=True)
        ex = jnp.exp(lg - m)
        probs = ex / jnp.sum(ex, axis=-1, keepdims=True)
        w, ind = _topk_w(probs)
        w_ref[0] = w
        sums_ref[2 * k:2 * k + 1, :] += jnp.sum(ind, axis=0, keepdims=True)
        sums_ref[2 * k + 1:2 * k + 2, :] += jnp.sum(probs, axis=0, keepdims=True)

    @pl.when(b == _B - 1)
    def _():
        f1 = sums_ref[0:1, :] / (_B * _N * _K) * (sums_ref[1:2, :] / (_B * _N))
        f2 = sums_ref[2:3, :] / (_B * _N * _K) * (sums_ref[3:4, :] / (_B * _N))
        lb_ref[...] = _E * jnp.concatenate(
            [jnp.sum(f1, axis=1, keepdims=True),
             jnp.sum(f2, axis=1, keepdims=True)], axis=1)


def _route2(gp1, gp2, h):
    gates = jnp.concatenate([gp1['gate'], gp2['gate']], axis=1)
    w1, w2, _, lb = pl.pallas_call(
        _route2_body,
        grid=(_B,),
        in_specs=[
            pl.BlockSpec((1, _N, _D), lambda b: (b, 0, 0)),
            pl.BlockSpec((_D, 2 * _E), lambda b: (0, 0)),
        ],
        out_specs=[
            pl.BlockSpec((1, _N, _E), lambda b: (b, 0, 0)),
            pl.BlockSpec((1, _N, _E), lambda b: (b, 0, 0)),
            pl.BlockSpec((8, _E), lambda b: (0, 0)),
            pl.BlockSpec((1, 2), lambda b: (0, 0)),
        ],
        out_shape=[
            jax.ShapeDtypeStruct((_B, _N, _E), _F32),
            jax.ShapeDtypeStruct((_B, _N, _E), _F32),
            jax.ShapeDtypeStruct((8, _E), _F32),
            jax.ShapeDtypeStruct((1, 2), _F32),
        ],
    )(h, gates)
    return w1, w2, lb[0, 0], lb[0, 1]


def _scores_pair(Whs, Whm, hb, adj, vec_ref, rbs_ref, rbm_ref, num_rel):
    srcs = jnp.dot(Whs, vec_ref[0, 0].reshape(_D, 1), preferred_element_type=_F32)
    dsts = jax.lax.dot_general(vec_ref[0, 1:2], Whs, (((1,), (1,)), ((), ())),
                               preferred_element_type=_F32)
    srcm = jnp.dot(Whm, vec_ref[0, 2].reshape(_D, 1), preferred_element_type=_F32)
    dstm = jax.lax.dot_general(vec_ref[0, 3:4], Whm, (((1,), (1,)), ((), ())),
                               preferred_element_type=_F32)
    bias_s = jnp.zeros((_N, _N), _F32)
    bias_m = jnp.zeros((_N, _N), _F32)
    for r in range(1, num_rel + 1):
        hit = adj == r
        bias_s = bias_s + jnp.where(hit, rbs_ref[0, 0, r], 0.0)
        bias_m = bias_m + jnp.where(hit, rbm_ref[0, 0, r], 0.0)
    mask = adj > 0
    anyrow = jnp.max(mask.astype(_F32), axis=1, keepdims=True)

    def finish(s):
        s = jnp.where(s >= 0, s, 0.2 * s)
        s = jnp.where(mask, s, -1e9)
        return _softmax_rows(s) * anyrow

    return finish(srcs + dsts + bias_s), finish(srcm + dstm + bias_m)


def _wcol(w_ref, e):
    lane = jax.lax.broadcasted_iota(jnp.int32, (_N, _E), 1)
    return jnp.sum(jnp.where(lane == e, w_ref[0], 0.0), axis=1, keepdims=True)


def _acc(e, ref, val):
    @pl.when(e == 0)
    def _():
        ref[0] = val

    @pl.when(e != 0)
    def _():
        ref[0] += val

    @pl.when(e == _E - 1)
    def _():
        ref[0] = _elu(ref[0])


def _moe2_body(h_ref, b1_ref, b2_ref, adj_ref, ws_ref, wm_ref,
               Ws_ref, Wm_ref, Wv_ref, vec_ref, rbs_ref, rbm_ref,
               os_ref, o1_ref, o2_ref, *, num_rel):
    e = pl.program_id(1)
    hb = _bf(h_ref[0])
    Whs = jnp.dot(hb, Ws_ref[0], preferred_element_type=_F32)
    Whm = jnp.dot(hb, Wm_ref[0], preferred_element_type=_F32)
    att_s, att_m = _scores_pair(Whs, Whm, hb, adj_ref[0], vec_ref,
                                rbs_ref, rbm_ref, num_rel)
    V1 = jnp.dot(_bf(b1_ref[0]), Wv_ref[0], preferred_element_type=_F32)
    V2 = jnp.dot(_bf(b2_ref[0]), Wv_ref[0], preferred_element_type=_F32)
    atts_w = _bf(_wcol(ws_ref, e) * att_s)
    attm_w = _bf(_wcol(wm_ref, e) * att_m)
    _acc(e, os_ref, jnp.dot(atts_w, _bf(Whs), preferred_element_type=_F32))
    _acc(e, o1_ref, jnp.dot(attm_w, _bf(V1), preferred_element_type=_F32))
    _acc(e, o2_ref, jnp.dot(attm_w, _bf(V2), preferred_element_type=_F32))


def _moe2(gps, gpm, h, b1, b2, adj, ws, wm, num_rel):
    R = num_rel + 1
    vecs = jnp.stack([gps['a_src'], gps['a_dst'],
                      gpm['a_src'], gpm['a_dst']], axis=1)
    os_, o1, o2 = pl.pallas_call(
        functools.partial(_moe2_body, num_rel=num_rel),
        grid=(_B, _E),
        in_specs=[
            pl.BlockSpec((1, _N, _D), lambda b, e: (b, 0, 0)),
            pl.BlockSpec((1, _N, _D), lambda b, e: (b, 0, 0)),
            pl.BlockSpec((1, _N, _D), lambda b, e: (b, 0, 0)),
            pl.BlockSpec((1, _N, _N), lambda b, e: (b, 0, 0)),
            pl.BlockSpec((1, _N, _E), lambda b, e: (b, 0, 0)),
            pl.BlockSpec((1, _N, _E), lambda b, e: (b, 0, 0)),
            pl.BlockSpec((1, _D, _D), lambda b, e: (e, 0, 0)),
            pl.BlockSpec((1, _D, _D), lambda b, e: (e, 0, 0)),
            pl.BlockSpec((1, _D, _D), lambda b, e: (e, 0, 0)),
            pl.BlockSpec((1, 4, _D), lambda b, e: (e, 0, 0)),
            pl.BlockSpec((1, 1, R), lambda b, e: (e, 0, 0)),
            pl.BlockSpec((1, 1, R), lambda b, e: (e, 0, 0)),
        ],
        out_specs=[pl.BlockSpec((1, _N, _D), lambda b, e: (b, 0, 0))] * 3,
        out_shape=[jax.ShapeDtypeStruct((_B, _N, _D), _F32)] * 3,
        compiler_params=pltpu.CompilerParams(
            dimension_semantics=("arbitrary", "arbitrary")),
    )(h, b1, b2, adj, ws, wm, _bf(gps['W']), _bf(gpm['W']), _bf(gpm['Wv']),
      vecs, gps['rel_bias'].reshape(_E, 1, R), gpm['rel_bias'].reshape(_E, 1, R))
    return os_, o1, o2


def _affine6_body(h1_ref, h2_ref, h3_ref, h4_ref, h5_ref, h6_ref, wf_ref,
                  o1_ref, o2_ref, o3_ref, o4_ref, o5_ref, o6_ref):
    hs = [_bf(r[0]) for r in (h1_ref, h2_ref, h3_ref, h4_ref, h5_ref, h6_ref)]
    outs = (o1_ref, o2_ref, o3_ref, o4_ref, o5_ref, o6_ref)
    pairs = ((0, 1), (1, 0), (2, 3), (3, 2), (4, 5), (5, 4))
    for i, (ia, ib) in enumerate(pairs):
        a, bm = hs[ia], hs[ib]
        proj = _bf(jnp.dot(a, wf_ref[i], preferred_element_type=_F32))
        s = jax.lax.dot_general(proj, bm, (((1,), (1,)), ((), ())),
                                preferred_element_type=_F32)
        att = _bf(_softmax_rows(s))
        outs[i][0] = jnp.dot(att, bm, preferred_element_type=_F32)


def _affine6(hs, wfs):
    wf = jnp.stack(wfs, axis=0)
    return pl.pallas_call(
        _affine6_body,
        grid=(_B,),
        in_specs=[pl.BlockSpec((1, _N, _D), lambda b: (b, 0, 0))] * 6 + [
            pl.BlockSpec((6, _D, _D), lambda b: (0, 0, 0)),
        ],
        out_specs=[pl.BlockSpec((1, _N, _D), lambda b: (b, 0, 0))] * 6,
        out_shape=[jax.ShapeDtypeStruct((_B, _N, _D), _F32)] * 6,
    )(*hs, _bf(wf))


def _mlp_body(f1_ref, f2_ref, f3_ref, f4_ref, f5_ref, f6_ref,
              u_ref, a_ref, v_ref, w1_ref, w2_ref, w3_ref,
              b1_ref, b2_ref, b3_ref, out_ref):
    parts = [_bf(f1_ref[0]), _bf(f2_ref[0]), _bf(f3_ref[0]), _bf(f4_ref[0]),
             _bf(f5_ref[0]), _bf(f6_ref[0]), u_ref[0], a_ref[0], v_ref[0]]
    offs = [0, _D, 2 * _D, 3 * _D, 4 * _D, 5 * _D,
            6 * _D, 6 * _D + _EMB, 6 * _D + _EMB + _AEMB]
    x = b1_ref[...]
    for part, off in zip(parts, offs):
        x = x + jnp.dot(part, w1_ref[off:off + part.shape[1], :],
                        preferred_element_type=_F32)
    x = _bf(jnp.maximum(x, 0.0))
    x = _bf(jnp.maximum(jnp.dot(x, w2_ref[...], preferred_element_type=_F32)
                        + b2_ref[...], 0.0))
    out_ref[0] = jnp.dot(x, w3_ref[...], preferred_element_type=_F32) + b3_ref[...]


def _mlp(feats, u, a, v, p):
    fin = 6 * _D + _EMB + _AEMB + _VEMB
    out = pl.pallas_call(
        _mlp_body,
        grid=(_B,),
        in_specs=[pl.BlockSpec((1, _N, _D), lambda b: (b, 0, 0))] * 6 + [
            pl.BlockSpec((1, _N, _EMB), lambda b: (b, 0, 0)),
            pl.BlockSpec((1, _N, _AEMB), lambda b: (b, 0, 0)),
            pl.BlockSpec((1, _N, _VEMB), lambda b: (b, 0, 0)),
            pl.BlockSpec((fin, _D), lambda b: (0, 0)),
            pl.BlockSpec((_D, _D), lambda b: (0, 0)),
            pl.BlockSpec((_D, _NCLS), lambda b: (0, 0)),
            pl.BlockSpec((1, _D), lambda b: (0, 0)),
            pl.BlockSpec((1, _D), lambda b: (0, 0)),
            pl.BlockSpec((1, _NCLS), lambda b: (0, 0)),
        ],
        out_specs=pl.BlockSpec((1, _N, _NCLS), lambda b: (b, 0, 0)),
        out_shape=jax.ShapeDtypeStruct((_B, _N, _NCLS), _F32),
    )(*feats, _bf(u), _bf(a), _bf(v),
      _bf(p['mlp_w'][0]), _bf(p['mlp_w'][1]), _bf(p['mlp_w'][2]),
      p['mlp_b'][0].reshape(1, _D), p['mlp_b'][1].reshape(1, _D),
      p['mlp_b'][2].reshape(1, _NCLS))
    return out


def kernel(utterance_features, audio_features, visual_features,
           semantic_adj, structure_adj, params):
    p = params
    h_t, h_a, h_v = _proj(utterance_features, audio_features,
                          visual_features, p)
    H = [[h_t, h_a, h_v]]
    lb_total = jnp.float32(0.0)
    for l in range(_L):
        sem_src = H[0] if l == 0 else H[2 * l - 1]
        stu_src = H[0] if l == 0 else H[2 * l]

        w_spk, w_ms, lb1, lb3 = _route2(p['spk'][l], p['mspk'][l], sem_src[0])
        w_dis, w_md, lb2, lb5 = _route2(p['dis'][l], p['mdis'][l], stu_src[0])
        h_sem, h_sta, h_stv = _moe2(p['spk'][l], p['mspk'][l], sem_src[0],
                                    sem_src[1], sem_src[2], semantic_adj,
                                    w_spk, w_ms, 6)
        h_stu, h_dta, h_dtv = _moe2(p['dis'][l], p['mdis'][l], stu_src[0],
                                    stu_src[1], stu_src[2], structure_adj,
                                    w_dis, w_md, 18)
        lb_total = lb_total + lb1 + lb2 + 2.0 * lb3 + 2.0 * lb5

        o1, o2, o7, o8, o9, o10 = _affine6(
            [h_sem, h_stu, h_sta, h_dta, h_stv, h_dtv],
            [p['affine1'], p['affine2'], p['affine7'], p['affine8'],
             p['affine9'], p['affine10']])
        H.append([o1, o7, o9])
        H.append([o2, o8, o10])

    feats = [H[-2][0], H[-1][0], H[-2][1], H[-1][1], H[-2][2], H[-1][2]]
    x = _mlp(feats, utterance_features, audio_features,
             visual_features, p)
    return x, lb_total
